# baseline (speedup 1.0000x reference)
import functools

import jax
import jax.numpy as jnp
from jax import lax
from jax.experimental import pallas as pl
from jax.experimental.pallas import tpu as pltpu
from jax.experimental.pallas import tpu_sc as plsc

N = 10000
D = 128
NC = 2
NS = 16
NW = NC * NS
L = 16
CHUNK = 80
NB = 4
N_PAD = 10240
RPT = N_PAD // NS


def _sc_degree(dst3, ch):
    mesh = plsc.VectorSubcoreMesh(core_axis_name="c", subcore_axis_name="s")

    @functools.partial(
        pl.kernel,
        out_type=jax.ShapeDtypeStruct((NC, N_PAD), jnp.float32),
        mesh=mesh,
        scratch_types=[
            pltpu.VMEM((ch, CHUNK), jnp.int32),
            pltpu.VMEM((CHUNK,), jnp.float32),
            pltpu.VMEM((RPT,), jnp.float32),
            pltpu.VMEM_SHARED((N_PAD,), jnp.float32),
            pltpu.SemaphoreType.DMA,
        ],
    )
    def deg_kernel(dst_hbm, out_hbm, idx_v, ones_v, zstage_v, acc_sh, sem):
        c = lax.axis_index("c")
        s = lax.axis_index("s")
        w = s * NC + c

        one = jnp.ones((L,), jnp.float32)
        zero = jnp.zeros((L,), jnp.float32)
        for i in range(CHUNK // L):
            ones_v[pl.ds(i * L, L)] = one

        def zinit(i, _):
            zstage_v[pl.ds(i * L, L)] = zero
            return ()
        lax.fori_loop(0, RPT // L, zinit, ())

        pltpu.sync_copy(zstage_v, acc_sh.at[pl.ds(s * RPT, RPT)])
        pltpu.sync_copy(dst_hbm.at[w], idx_v)
        plsc.subcore_barrier()

        def fire(j, _):
            pltpu.async_copy(ones_v, acc_sh.at[idx_v.at[j]], sem, add=True)
            return ()
        lax.fori_loop(0, ch, fire, ())

        def drain(j, _):
            pltpu.make_async_copy(ones_v, acc_sh.at[idx_v.at[j]], sem).wait()
            return ()
        lax.fori_loop(0, ch, drain, ())

        plsc.subcore_barrier()
        pltpu.sync_copy(acc_sh.at[pl.ds(s * RPT, RPT)],
                        out_hbm.at[c, pl.ds(s * RPT, RPT)])

    return deg_kernel(dst3)


WIN = 4
NBUF = WIN


def _sc_aggregate(hp, src4, dst4, nwin):
    mesh = plsc.VectorSubcoreMesh(core_axis_name="c", subcore_axis_name="s")

    @functools.partial(
        pl.kernel,
        out_type=jax.ShapeDtypeStruct((NC, N_PAD, D), jnp.float32),
        mesh=mesh,
        scratch_types=[
            pltpu.VMEM((2, WIN, CHUNK), jnp.int32),
            pltpu.VMEM((2, WIN, CHUNK), jnp.int32),
            pltpu.VMEM((CHUNK, D), jnp.float32),
            pltpu.VMEM((CHUNK, D), jnp.float32),
            pltpu.VMEM((CHUNK, D), jnp.float32),
            pltpu.VMEM((CHUNK, D), jnp.float32),
            pltpu.VMEM_SHARED((N_PAD, D), jnp.float32),
            pltpu.SemaphoreType.DMA,
            pltpu.SemaphoreType.DMA,
            pltpu.SemaphoreType.DMA,
        ],
    )
    def agg_kernel(hp_hbm, src_hbm, dst_hbm, out_hbm,
                   srcw, dstw, rows0, rows1, rows2, rows3,
                   acc_sh, gsem, ssem, isem):
        c = lax.axis_index("c")
        s = lax.axis_index("s")
        w = s * NC + c
        rows = (rows0, rows1, rows2, rows3)

        zero = jnp.zeros((L,), jnp.float32)

        def zrow(j, _):
            for k in range(D // L):
                rows0[j, pl.ds(k * L, L)] = zero
            return ()
        lax.fori_loop(0, CHUNK, zrow, ())
        for i in range(RPT // CHUNK):
            pltpu.sync_copy(rows0, acc_sh.at[pl.ds(s * RPT + i * CHUNK, CHUNK)])
        REM = RPT % CHUNK
        if REM:
            pltpu.sync_copy(
                rows0.at[pl.ds(0, REM)],
                acc_sh.at[pl.ds(s * RPT + (RPT // CHUNK) * CHUNK, REM)])

        pltpu.sync_copy(src_hbm.at[w, 0], srcw.at[0])
        pltpu.sync_copy(dst_hbm.at[w, 0], dstw.at[0])
        plsc.subcore_barrier()

        pltpu.async_copy(hp_hbm.at[srcw.at[0, 0]], rows0, gsem)
        pltpu.async_copy(hp_hbm.at[srcw.at[0, 1]], rows1, gsem)
        pltpu.async_copy(hp_hbm.at[srcw.at[0, 2]], rows2, gsem)

        def window(win, _):
            cur = lax.rem(win, 2)
            nxt = 1 - cur
            more = win + 1 < nwin

            @pl.when(more)
            def _():
                pltpu.async_copy(src_hbm.at[w, win + 1], srcw.at[nxt], isem)
                pltpu.async_copy(dst_hbm.at[w, win + 1], dstw.at[nxt], isem)

            for k in range(WIN):
                pltpu.make_async_copy(hp_hbm.at[srcw.at[cur, k]],
                                      rows[k % 4], gsem).wait()
                if k >= 1:
                    pltpu.make_async_copy(rows[(k - 1) % 4],
                                          acc_sh.at[dstw.at[cur, k - 1]],
                                          ssem).wait()
                kn = k + 3
                if kn < WIN:
                    pltpu.async_copy(hp_hbm.at[srcw.at[cur, kn]],
                                     rows[kn % 4], gsem)
                else:
                    if kn == WIN:
                        @pl.when(more)
                        def _():
                            pltpu.make_async_copy(src_hbm.at[w, win + 1],
                                                  srcw.at[nxt], isem).wait()
                            pltpu.make_async_copy(dst_hbm.at[w, win + 1],
                                                  dstw.at[nxt], isem).wait()

                    @pl.when(more)
                    def _():
                        pltpu.async_copy(hp_hbm.at[srcw.at[nxt, kn - WIN]],
                                         rows[kn % 4], gsem)
                pltpu.async_copy(rows[k % 4], acc_sh.at[dstw.at[cur, k]],
                                 ssem, add=True)
            pltpu.make_async_copy(rows[(WIN - 1) % 4],
                                  acc_sh.at[dstw.at[cur, WIN - 1]],
                                  ssem).wait()
            return ()
        lax.fori_loop(0, nwin, window, ())

        plsc.subcore_barrier()
        pltpu.sync_copy(acc_sh.at[pl.ds(s * RPT, RPT)],
                        out_hbm.at[c, pl.ds(s * RPT, RPT)])

    return agg_kernel(hp, src4, dst4)


def _tc_project(X, W, b, degcol):
    BLK = 2048
    grid = N_PAD // BLK

    def body(x_ref, w_ref, b_ref, deg_ref, hp_ref, dinv_ref):
        i = pl.program_id(0)
        h = jnp.dot(x_ref[...], w_ref[...],
                    preferred_element_type=jnp.float32) + b_ref[...]
        dinv = lax.rsqrt(deg_ref[...])
        rows = i * BLK + lax.broadcasted_iota(jnp.int32, (BLK, 1), 0)
        valid = rows < N
        dinv = jnp.where(valid, dinv, 0.0)
        hp_ref[...] = jnp.where(valid, h * dinv, 0.0)
        dinv_ref[...] = dinv

    return pl.pallas_call(
        body,
        grid=(grid,),
        in_specs=[
            pl.BlockSpec((BLK, D), lambda i: (i, 0)),
            pl.BlockSpec((D, D), lambda i: (0, 0)),
            pl.BlockSpec((1, D), lambda i: (0, 0)),
            pl.BlockSpec((BLK, 1), lambda i: (i, 0)),
        ],
        out_specs=[
            pl.BlockSpec((BLK, D), lambda i: (i, 0)),
            pl.BlockSpec((BLK, 1), lambda i: (i, 0)),
        ],
        out_shape=[
            jax.ShapeDtypeStruct((N_PAD, D), jnp.float32),
            jax.ShapeDtypeStruct((N_PAD, 1), jnp.float32),
        ],
    )(X, W, b, degcol)


def _tc_finish(parts, hp, dinv):
    BLK = 2000
    grid = N // BLK

    def body(p_ref, hp_ref, dinv_ref, o_ref):
        acc = p_ref[0] + p_ref[1] + hp_ref[...]
        o_ref[...] = jnp.maximum(acc * dinv_ref[...], 0.0)

    return pl.pallas_call(
        body,
        grid=(grid,),
        in_specs=[
            pl.BlockSpec((NC, BLK, D), lambda i: (0, i, 0)),
            pl.BlockSpec((BLK, D), lambda i: (i, 0)),
            pl.BlockSpec((BLK, 1), lambda i: (i, 0)),
        ],
        out_specs=pl.BlockSpec((BLK, D), lambda i: (i, 0)),
        out_shape=jax.ShapeDtypeStruct((N, D), jnp.float32),
    )(parts, hp, dinv)


def kernel(X, edge_index, W, b):
    E = edge_index.shape[1]
    epw = -(-E // NW)
    ch = -(-epw // CHUNK)
    ch = -(-ch // WIN) * WIN
    nwin = ch // WIN
    e_pad = NW * ch * CHUNK

    src = edge_index[0].astype(jnp.int32)
    dst = edge_index[1].astype(jnp.int32)
    pad = jnp.full((e_pad - E,), N, jnp.int32)
    src4 = jnp.concatenate([src, pad]).reshape(NW, nwin, WIN, CHUNK)
    dst4 = jnp.concatenate([dst, pad]).reshape(NW, nwin, WIN, CHUNK)

    degp = _sc_degree(dst4.reshape(NW, ch, CHUNK), ch)
    degcol = (degp[0] + degp[1] + 1.0)[:, None]
    hp, dinv = _tc_project(X, W, b.reshape(1, D), degcol)
    parts = _sc_aggregate(hp, src4, dst4, nwin)
    return _tc_finish(parts, hp, dinv)


# speedup vs baseline: 1.5541x; 1.5541x over previous
import functools

import jax
import jax.numpy as jnp
from jax import lax
from jax.experimental import pallas as pl
from jax.experimental.pallas import tpu as pltpu
from jax.experimental.pallas import tpu_sc as plsc

N = 10000
D = 128
NC = 2
NS = 16
NW = NC * NS
L = 16
CHUNK = 116
NB = 3
N_PAD = 10240
RPT = N_PAD // NS


def _sc_degree(dst3, ch):
    mesh = plsc.VectorSubcoreMesh(core_axis_name="c", subcore_axis_name="s")

    @functools.partial(
        pl.kernel,
        out_type=jax.ShapeDtypeStruct((NC, N_PAD), jnp.float32),
        mesh=mesh,
        scratch_types=[
            pltpu.VMEM((ch, CHUNK), jnp.int32),
            pltpu.VMEM((CHUNK,), jnp.float32),
            pltpu.VMEM((RPT,), jnp.float32),
            pltpu.VMEM_SHARED((N_PAD,), jnp.float32),
            pltpu.SemaphoreType.DMA,
        ],
    )
    def deg_kernel(dst_hbm, out_hbm, idx_v, ones_v, zstage_v, acc_sh, sem):
        c = lax.axis_index("c")
        s = lax.axis_index("s")
        w = s * NC + c

        one = jnp.ones((L,), jnp.float32)
        zero = jnp.zeros((L,), jnp.float32)
        for i in range(CHUNK // L):
            ones_v[pl.ds(i * L, L)] = one

        def zinit(i, _):
            zstage_v[pl.ds(i * L, L)] = zero
            return ()
        lax.fori_loop(0, RPT // L, zinit, ())

        pltpu.sync_copy(zstage_v, acc_sh.at[pl.ds(s * RPT, RPT)])
        pltpu.sync_copy(dst_hbm.at[w], idx_v)
        plsc.subcore_barrier()

        def fire(j, _):
            pltpu.async_copy(ones_v, acc_sh.at[idx_v.at[j]], sem, add=True)
            return ()
        lax.fori_loop(0, ch, fire, ())

        def drain(j, _):
            pltpu.make_async_copy(ones_v, acc_sh.at[idx_v.at[j]], sem).wait()
            return ()
        lax.fori_loop(0, ch, drain, ())

        plsc.subcore_barrier()
        pltpu.sync_copy(acc_sh.at[pl.ds(s * RPT, RPT)],
                        out_hbm.at[c, pl.ds(s * RPT, RPT)])

    return deg_kernel(dst3)


WIN = 3
NBUF = WIN


def _sc_aggregate(hp, src4, dst4, nwin):
    mesh = plsc.VectorSubcoreMesh(core_axis_name="c", subcore_axis_name="s")

    @functools.partial(
        pl.kernel,
        out_type=jax.ShapeDtypeStruct((NC, N_PAD, D), jnp.float32),
        mesh=mesh,
        scratch_types=[
            pltpu.VMEM((2, WIN, CHUNK), jnp.int32),
            pltpu.VMEM((2, WIN, CHUNK), jnp.int32),
            pltpu.VMEM((CHUNK, D), jnp.float32),
            pltpu.VMEM((CHUNK, D), jnp.float32),
            pltpu.VMEM((CHUNK, D), jnp.float32),
            pltpu.VMEM_SHARED((N_PAD, D), jnp.float32),
            pltpu.SemaphoreType.DMA,
            pltpu.SemaphoreType.DMA,
            pltpu.SemaphoreType.DMA,
        ],
    )
    def agg_kernel(hp_hbm, src_hbm, dst_hbm, out_hbm,
                   srcw, dstw, rows0, rows1, rows2,
                   acc_sh, gsem, ssem, isem):
        c = lax.axis_index("c")
        s = lax.axis_index("s")
        w = s * NC + c
        rows = (rows0, rows1, rows2)

        zero = jnp.zeros((L,), jnp.float32)

        def zrow(j, _):
            for k in range(D // L):
                rows0[j, pl.ds(k * L, L)] = zero
            return ()
        lax.fori_loop(0, CHUNK, zrow, ())
        for i in range(RPT // CHUNK):
            pltpu.sync_copy(rows0, acc_sh.at[pl.ds(s * RPT + i * CHUNK, CHUNK)])
        REM = RPT % CHUNK
        if REM:
            pltpu.sync_copy(
                rows0.at[pl.ds(0, REM)],
                acc_sh.at[pl.ds(s * RPT + (RPT // CHUNK) * CHUNK, REM)])

        pltpu.sync_copy(src_hbm.at[w, 0], srcw.at[0])
        pltpu.sync_copy(dst_hbm.at[w, 0], dstw.at[0])
        plsc.subcore_barrier()

        pltpu.async_copy(hp_hbm.at[srcw.at[0, 0]], rows0, gsem)
        pltpu.async_copy(hp_hbm.at[srcw.at[0, 1]], rows1, gsem)

        def window(win, _):
            cur = lax.rem(win, 2)
            nxt = 1 - cur
            more = win + 1 < nwin

            @pl.when(more)
            def _():
                pltpu.async_copy(src_hbm.at[w, win + 1], srcw.at[nxt], isem)
                pltpu.async_copy(dst_hbm.at[w, win + 1], dstw.at[nxt], isem)

            for k in range(WIN):
                pltpu.make_async_copy(hp_hbm.at[srcw.at[cur, k]],
                                      rows[k % 3], gsem).wait()
                if k >= 1:
                    pltpu.make_async_copy(rows[(k - 1) % 3],
                                          acc_sh.at[dstw.at[cur, k - 1]],
                                          ssem).wait()
                kn = k + 2
                if kn < WIN:
                    pltpu.async_copy(hp_hbm.at[srcw.at[cur, kn]],
                                     rows[kn % 3], gsem)
                else:
                    if kn == WIN:
                        @pl.when(more)
                        def _():
                            pltpu.make_async_copy(src_hbm.at[w, win + 1],
                                                  srcw.at[nxt], isem).wait()
                            pltpu.make_async_copy(dst_hbm.at[w, win + 1],
                                                  dstw.at[nxt], isem).wait()

                    @pl.when(more)
                    def _():
                        pltpu.async_copy(hp_hbm.at[srcw.at[nxt, kn - WIN]],
                                         rows[kn % 3], gsem)
                pltpu.async_copy(rows[k % 3], acc_sh.at[dstw.at[cur, k]],
                                 ssem, add=True)
            pltpu.make_async_copy(rows[(WIN - 1) % 3],
                                  acc_sh.at[dstw.at[cur, WIN - 1]],
                                  ssem).wait()
            return ()
        lax.fori_loop(0, nwin, window, ())

        plsc.subcore_barrier()
        pltpu.sync_copy(acc_sh.at[pl.ds(s * RPT, RPT)],
                        out_hbm.at[c, pl.ds(s * RPT, RPT)])

    return agg_kernel(hp, src4, dst4)


def _tc_project(X, W, b, degcol):
    BLK = 2048
    grid = N_PAD // BLK

    def body(x_ref, w_ref, b_ref, deg_ref, hp_ref, dinv_ref):
        i = pl.program_id(0)
        h = jnp.dot(x_ref[...], w_ref[...],
                    preferred_element_type=jnp.float32) + b_ref[...]
        dinv = lax.rsqrt(deg_ref[...])
        rows = i * BLK + lax.broadcasted_iota(jnp.int32, (BLK, 1), 0)
        valid = rows < N
        dinv = jnp.where(valid, dinv, 0.0)
        hp_ref[...] = jnp.where(valid, h * dinv, 0.0)
        dinv_ref[...] = dinv

    return pl.pallas_call(
        body,
        grid=(grid,),
        in_specs=[
            pl.BlockSpec((BLK, D), lambda i: (i, 0)),
            pl.BlockSpec((D, D), lambda i: (0, 0)),
            pl.BlockSpec((1, D), lambda i: (0, 0)),
            pl.BlockSpec((BLK, 1), lambda i: (i, 0)),
        ],
        out_specs=[
            pl.BlockSpec((BLK, D), lambda i: (i, 0)),
            pl.BlockSpec((BLK, 1), lambda i: (i, 0)),
        ],
        out_shape=[
            jax.ShapeDtypeStruct((N_PAD, D), jnp.float32),
            jax.ShapeDtypeStruct((N_PAD, 1), jnp.float32),
        ],
    )(X, W, b, degcol)


def _tc_finish(parts, hp, dinv):
    BLK = 2000
    grid = N // BLK

    def body(p_ref, hp_ref, dinv_ref, o_ref):
        acc = p_ref[0] + p_ref[1] + hp_ref[...]
        o_ref[...] = jnp.maximum(acc * dinv_ref[...], 0.0)

    return pl.pallas_call(
        body,
        grid=(grid,),
        in_specs=[
            pl.BlockSpec((NC, BLK, D), lambda i: (0, i, 0)),
            pl.BlockSpec((BLK, D), lambda i: (i, 0)),
            pl.BlockSpec((BLK, 1), lambda i: (i, 0)),
        ],
        out_specs=pl.BlockSpec((BLK, D), lambda i: (i, 0)),
        out_shape=jax.ShapeDtypeStruct((N, D), jnp.float32),
    )(parts, hp, dinv)


def kernel(X, edge_index, W, b):
    E = edge_index.shape[1]
    epw = -(-E // NW)
    ch = -(-epw // CHUNK)
    ch = -(-ch // WIN) * WIN
    nwin = ch // WIN
    e_pad = NW * ch * CHUNK

    src = edge_index[0].astype(jnp.int32)
    dst = edge_index[1].astype(jnp.int32)
    pad = jnp.full((e_pad - E,), N, jnp.int32)
    src4 = jnp.concatenate([src, pad]).reshape(NW, nwin, WIN, CHUNK)
    dst4 = jnp.concatenate([dst, pad]).reshape(NW, nwin, WIN, CHUNK)

    degp = _sc_degree(dst4.reshape(NW, ch, CHUNK), ch)
    degcol = (degp[0] + degp[1] + 1.0)[:, None]
    hp, dinv = _tc_project(X, W, b.reshape(1, D), degcol)
    parts = _sc_aggregate(hp, src4, dst4, nwin)
    return _tc_finish(parts, hp, dinv)
